# baseline (speedup 1.0000x reference)
import functools

import jax
import jax.numpy as jnp
from jax import lax
from jax.experimental import pallas as pl
from jax.experimental.pallas import tpu as pltpu
from jax.experimental.pallas import tpu_sc as plsc

_N = 10000
_E = 320000
_DH = 128
_DW1 = _DH + 16
_NC = 2
_NS = 16
_NW = _NC * _NS
_CHUNK = 80
_NCHUNK = 126
_EPW = _NCHUNK * _CHUNK
_EPAD = _NW * _EPW
_NA = _N + 16
_RPT = _N // _NS


def _sc_agg_body(h_hbm, edges_hbm, zsum_hbm, sum_out, acc, idxb,
                 rows0, rows1, rows2, g0, g1, g2, s0, s1, s2, r0s, r1s, r2s):
    cid = lax.axis_index("c")
    sid = lax.axis_index("s")
    wid = sid * _NC + cid
    row0 = sid * _RPT
    rows = (rows0, rows1, rows2)
    gsem = (g0, g1, g2)
    ssem = (s0, s1, s2)
    rsem = (r0s, r1s, r2s)
    T = _NCHUNK // 3

    def srow(tb, j):
        return idxb.at[tb, 0, j]

    def drow(tb, j):
        return idxb.at[tb, 1, j]

    def refill_start(t, tb):
        pltpu.make_async_copy(edges_hbm.at[0, wid, pl.ds(3 * t, 3)],
                              idxb.at[tb, 0], rsem[tb]).start()
        pltpu.make_async_copy(edges_hbm.at[1, wid, pl.ds(3 * t, 3)],
                              idxb.at[tb, 1], rsem[tb]).start()

    def refill_wait(t, tb):
        pltpu.make_async_copy(edges_hbm.at[0, wid, pl.ds(3 * t, 3)],
                              idxb.at[tb, 0], rsem[tb]).wait()
        pltpu.make_async_copy(edges_hbm.at[1, wid, pl.ds(3 * t, 3)],
                              idxb.at[tb, 1], rsem[tb]).wait()

    def step(buf, gw_row, sc_row, nx_row, first=False):
        pltpu.make_async_copy(h_hbm.at[gw_row], rows[buf], gsem[buf]).wait()
        pltpu.make_async_copy(rows[buf], acc.at[sc_row],
                              ssem[buf]).start(add=True)
        pbuf = (buf - 1) % 3
        if not first:
            pltpu.make_async_copy(rows[pbuf], acc.at[sc_row],
                                  ssem[pbuf]).wait()
        if nx_row is not None:
            pltpu.make_async_copy(h_hbm.at[nx_row], rows[pbuf],
                                  gsem[pbuf]).start()

    pltpu.sync_copy(zsum_hbm.at[pl.ds(row0, _RPT)],
                    acc.at[pl.ds(row0, _RPT)])
    for a in range(2):
        pltpu.sync_copy(edges_hbm.at[a, wid, pl.ds(0, 3)], idxb.at[0, a])
        pltpu.sync_copy(edges_hbm.at[a, wid, pl.ds(3, 3)], idxb.at[1, a])
    refill_start(2, 2)
    pltpu.async_copy(h_hbm.at[srow(0, 0)], rows0, g0)
    pltpu.async_copy(h_hbm.at[srow(0, 1)], rows1, g1)
    plsc.subcore_barrier()

    step(0, srow(0, 0), drow(0, 0), srow(0, 2), first=True)
    step(1, srow(0, 1), drow(0, 1), srow(1, 0))
    step(2, srow(0, 2), drow(0, 2), srow(1, 1))

    def triple(t, tb):
        nb = (tb + 1) % 3
        fb = (tb + 2) % 3
        refill_wait(t + 1, nb)
        step(0, srow(tb, 0), drow(tb, 0), srow(tb, 2))
        refill_start(t + 2, fb)
        step(1, srow(tb, 1), drow(tb, 1), srow(nb, 0))
        step(2, srow(tb, 2), drow(tb, 2), srow(nb, 1))

    @pl.loop(1, T - 2, step=3)
    def _(t):
        triple(t, 1)
        triple(t + 1, 2)
        triple(t + 2, 0)

    tb = (T - 2) % 3
    nb = (T - 1) % 3
    refill_wait(T - 1, nb)
    step(0, srow(tb, 0), drow(tb, 0), srow(tb, 2))
    step(1, srow(tb, 1), drow(tb, 1), srow(nb, 0))
    step(2, srow(tb, 2), drow(tb, 2), srow(nb, 1))
    step(0, srow(nb, 0), drow(nb, 0), srow(nb, 2))
    step(1, srow(nb, 1), drow(nb, 1), None)
    step(2, srow(nb, 2), drow(nb, 2), None)
    pltpu.make_async_copy(rows2, acc.at[drow(nb, 2)], ssem[2]).wait()

    plsc.subcore_barrier()
    pltpu.sync_copy(acc.at[pl.ds(row0, _RPT)],
                    sum_out.at[cid, pl.ds(row0, _RPT)])


def _sc_aggregate(h, edges):
    dw = h.shape[1]
    mesh = plsc.VectorSubcoreMesh(core_axis_name="c", subcore_axis_name="s")
    zsum = jnp.zeros((_N, dw), jnp.float32)
    k = pl.kernel(
        _sc_agg_body,
        out_type=jax.ShapeDtypeStruct((_NC, _N, dw), jnp.float32),
        mesh=mesh,
        scratch_types=[
            pltpu.VMEM_SHARED((_NA, dw), jnp.float32),
            pltpu.VMEM((3, 2, 3, _CHUNK), jnp.int32),
            pltpu.VMEM((_CHUNK, dw), jnp.float32),
            pltpu.VMEM((_CHUNK, dw), jnp.float32),
            pltpu.VMEM((_CHUNK, dw), jnp.float32),
            pltpu.SemaphoreType.DMA,
            pltpu.SemaphoreType.DMA,
            pltpu.SemaphoreType.DMA,
            pltpu.SemaphoreType.DMA,
            pltpu.SemaphoreType.DMA,
            pltpu.SemaphoreType.DMA,
            pltpu.SemaphoreType.DMA,
            pltpu.SemaphoreType.DMA,
            pltpu.SemaphoreType.DMA,
        ],
        compiler_params=pltpu.CompilerParams(use_tc_tiling_on_sc=False),
    )
    return k(h, edges, zsum)


def _encode_body(x_ref, wi_ref, bi_ref, q_ref, wq_ref, bq_ref, oc_ref,
                 o_ref):
    q = jnp.dot(q_ref[...], wq_ref[...],
                preferred_element_type=jnp.float32) + bq_ref[...]
    h = jnp.dot(x_ref[...], wi_ref[...],
                preferred_element_type=jnp.float32) + bi_ref[...] + q
    o_ref[...] = jnp.concatenate(
        [h, jnp.broadcast_to(oc_ref[...], (_N, _DW1 - _DH))], axis=-1)


def _norm_residual(h, mean, wl_ref, bl_ref, wr_ref, g_ref, be_ref):
    hout = (jnp.dot(mean, wl_ref[...], preferred_element_type=jnp.float32)
            + bl_ref[...]
            + jnp.dot(h, wr_ref[...], preferred_element_type=jnp.float32))
    m = jnp.mean(hout, axis=-1, keepdims=True)
    d = hout - m
    var = jnp.mean(d * d, axis=-1, keepdims=True)
    y = d * lax.rsqrt(var + 1e-5) * g_ref[...] + be_ref[...]
    return h + y


def _layer0_body(hx_ref, s_ref, wl_ref, bl_ref, wr_ref, g_ref, be_ref,
                 o_ref, c_ref):
    s = s_ref[0] + s_ref[1]
    cnt = jnp.maximum(s[:, _DH:_DH + 1], 1.0)
    mean = s[:, :_DH] / cnt
    h = hx_ref[:, :_DH]
    hn = _norm_residual(h, mean, wl_ref, bl_ref, wr_ref, g_ref, be_ref)
    o_ref[...] = jnp.maximum(hn, 0.0)
    c_ref[...] = cnt


def _final_body(h_ref, s_ref, c_ref, wl_ref, bl_ref, wr_ref, g_ref,
                be_ref, watt_ref, batt_ref, o_ref, a_ref):
    mean = (s_ref[0] + s_ref[1]) / c_ref[...]
    hn = _norm_residual(h_ref[...], mean, wl_ref, bl_ref, wr_ref, g_ref,
                        be_ref)
    o_ref[...] = hn
    logits = jnp.dot(hn, watt_ref[...],
                     preferred_element_type=jnp.float32) + batt_ref[...]
    z = logits - jnp.max(logits, axis=0, keepdims=True)
    e = jnp.exp(z)
    a_ref[...] = e / jnp.sum(e, axis=0, keepdims=True)


def kernel(x, edge_index, edge_attr, query_embedding, W_in, b_in, W_q, b_q,
           Wl0, bl0, Wr0, g0, be0, Wl1, bl1, Wr1, g1, be1, W_att, b_att):
    del edge_attr
    f32 = jnp.float32
    i32 = jnp.int32
    sds = jax.ShapeDtypeStruct

    npad = _EPAD - _E
    pad = jnp.concatenate([jnp.zeros((1, npad), i32),
                           jnp.full((1, npad), _N, i32)], axis=0)
    edges = jnp.concatenate([edge_index, pad],
                            axis=1).reshape(2, _NW, _NCHUNK, _CHUNK)

    onescol = jnp.zeros((1, _DW1 - _DH), f32).at[0, 0].set(1.0)
    hx = pl.pallas_call(
        _encode_body, out_shape=sds((_N, _DW1), f32))(
            x, W_in, b_in.reshape(1, _DH), query_embedding.reshape(1, -1),
            W_q, b_q.reshape(1, _DH), onescol)

    sums0 = _sc_aggregate(hx, edges)

    h1, cnt = pl.pallas_call(
        _layer0_body, out_shape=(sds((_N, _DH), f32), sds((_N, 1), f32)))(
            hx, sums0, Wl0, bl0.reshape(1, -1), Wr0,
            g0.reshape(1, -1), be0.reshape(1, -1))

    sums1 = _sc_aggregate(h1, edges)

    h2, attn = pl.pallas_call(
        _final_body, out_shape=(sds((_N, _DH), f32), sds((_N, 1), f32)))(
            h1, sums1, cnt, Wl1, bl1.reshape(1, -1), Wr1,
            g1.reshape(1, -1), be1.reshape(1, -1), W_att,
            b_att.reshape(1, 1))

    return h2, attn.reshape(-1)


# speedup vs baseline: 1.1058x; 1.1058x over previous
import functools

import jax
import jax.numpy as jnp
from jax import lax
from jax.experimental import pallas as pl
from jax.experimental.pallas import tpu as pltpu
from jax.experimental.pallas import tpu_sc as plsc

_N = 10000
_E = 320000
_DH = 128
_CW = 16
_NC = 2
_NS = 16
_NW = _NC * _NS
_CHUNK = 80
_NCHUNK = 126
_EPW = _NCHUNK * _CHUNK
_EPAD = _NW * _EPW
_NA = _N + 16
_RPT = _N // _NS


def _sc_agg_body(with_cnt, h_hbm, edges_hbm, zsum_hbm, zcnt_hbm, ones_hbm,
                 sum_out, cnt_out, acc, cnt, idxb, rows0, rows1, rows2,
                 ones_v, g0, g1, g2, s0, s1, s2, r0s, r1s, r2s):
    cid = lax.axis_index("c")
    sid = lax.axis_index("s")
    wid = sid * _NC + cid
    row0 = sid * _RPT
    rows = (rows0, rows1, rows2)
    gsem = (g0, g1, g2)
    ssem = (s0, s1, s2)
    rsem = (r0s, r1s, r2s)
    T = _NCHUNK // 3

    def srow(tb, j):
        return idxb.at[tb, 0, j]

    def drow(tb, j):
        return idxb.at[tb, 1, j]

    def refill_start(t, tb):
        pltpu.make_async_copy(edges_hbm.at[0, wid, pl.ds(3 * t, 3)],
                              idxb.at[tb, 0], rsem[tb]).start()
        pltpu.make_async_copy(edges_hbm.at[1, wid, pl.ds(3 * t, 3)],
                              idxb.at[tb, 1], rsem[tb]).start()

    def refill_wait(t, tb):
        pltpu.make_async_copy(edges_hbm.at[0, wid, pl.ds(3 * t, 3)],
                              idxb.at[tb, 0], rsem[tb]).wait()
        pltpu.make_async_copy(edges_hbm.at[1, wid, pl.ds(3 * t, 3)],
                              idxb.at[tb, 1], rsem[tb]).wait()

    def step(buf, gw_row, sc_row, nx_row, first=False):
        pltpu.make_async_copy(h_hbm.at[gw_row], rows[buf], gsem[buf]).wait()
        pltpu.make_async_copy(rows[buf], acc.at[sc_row],
                              ssem[buf]).start(add=True)
        if with_cnt:
            pltpu.make_async_copy(ones_v, cnt.at[sc_row],
                                  ssem[buf]).start(add=True)
        pbuf = (buf - 1) % 3
        if not first:
            pltpu.make_async_copy(rows[pbuf], acc.at[sc_row],
                                  ssem[pbuf]).wait()
            if with_cnt:
                pltpu.make_async_copy(ones_v, cnt.at[sc_row],
                                      ssem[pbuf]).wait()
        if nx_row is not None:
            pltpu.make_async_copy(h_hbm.at[nx_row], rows[pbuf],
                                  gsem[pbuf]).start()

    pltpu.sync_copy(zsum_hbm.at[pl.ds(row0, _RPT)],
                    acc.at[pl.ds(row0, _RPT)])
    if with_cnt:
        pltpu.sync_copy(zcnt_hbm.at[pl.ds(row0, _RPT)],
                        cnt.at[pl.ds(row0, _RPT)])
        pltpu.sync_copy(ones_hbm, ones_v)
    for a in range(2):
        pltpu.sync_copy(edges_hbm.at[a, wid, pl.ds(0, 3)], idxb.at[0, a])
        pltpu.sync_copy(edges_hbm.at[a, wid, pl.ds(3, 3)], idxb.at[1, a])
    refill_start(2, 2)
    pltpu.async_copy(h_hbm.at[srow(0, 0)], rows0, g0)
    pltpu.async_copy(h_hbm.at[srow(0, 1)], rows1, g1)
    plsc.subcore_barrier()

    step(0, srow(0, 0), drow(0, 0), srow(0, 2), first=True)
    step(1, srow(0, 1), drow(0, 1), srow(1, 0))
    step(2, srow(0, 2), drow(0, 2), srow(1, 1))

    def triple(t, tb):
        nb = (tb + 1) % 3
        fb = (tb + 2) % 3
        refill_wait(t + 1, nb)
        step(0, srow(tb, 0), drow(tb, 0), srow(tb, 2))
        refill_start(t + 2, fb)
        step(1, srow(tb, 1), drow(tb, 1), srow(nb, 0))
        step(2, srow(tb, 2), drow(tb, 2), srow(nb, 1))

    @pl.loop(1, T - 2, step=3)
    def _(t):
        triple(t, 1)
        triple(t + 1, 2)
        triple(t + 2, 0)

    tb = (T - 2) % 3
    nb = (T - 1) % 3
    refill_wait(T - 1, nb)
    step(0, srow(tb, 0), drow(tb, 0), srow(tb, 2))
    step(1, srow(tb, 1), drow(tb, 1), srow(nb, 0))
    step(2, srow(tb, 2), drow(tb, 2), srow(nb, 1))
    step(0, srow(nb, 0), drow(nb, 0), srow(nb, 2))
    step(1, srow(nb, 1), drow(nb, 1), None)
    step(2, srow(nb, 2), drow(nb, 2), None)
    pltpu.make_async_copy(rows2, acc.at[drow(nb, 2)], ssem[2]).wait()
    if with_cnt:
        pltpu.make_async_copy(ones_v, cnt.at[drow(nb, 2)], ssem[2]).wait()

    plsc.subcore_barrier()
    pltpu.sync_copy(acc.at[pl.ds(row0, _RPT)],
                    sum_out.at[cid, pl.ds(row0, _RPT)])
    if with_cnt:
        pltpu.sync_copy(cnt.at[pl.ds(row0, _RPT)],
                        cnt_out.at[cid, pl.ds(row0, _RPT)])


def _sc_aggregate(h, edges, with_cnt):
    mesh = plsc.VectorSubcoreMesh(core_axis_name="c", subcore_axis_name="s")
    zsum = jnp.zeros((_N, _DH), jnp.float32)
    zcnt = jnp.zeros((_N, _CW), jnp.float32)
    ones = jnp.ones((_CHUNK, _CW), jnp.float32)
    k = pl.kernel(
        functools.partial(_sc_agg_body, with_cnt),
        out_type=(jax.ShapeDtypeStruct((_NC, _N, _DH), jnp.float32),
                  jax.ShapeDtypeStruct((_NC, _N, _CW), jnp.float32)),
        mesh=mesh,
        scratch_types=[
            pltpu.VMEM_SHARED((_NA, _DH), jnp.float32),
            pltpu.VMEM_SHARED((_NA, _CW) if with_cnt else (8, _CW),
                              jnp.float32),
            pltpu.VMEM((3, 2, 3, _CHUNK), jnp.int32),
            pltpu.VMEM((_CHUNK, _DH), jnp.float32),
            pltpu.VMEM((_CHUNK, _DH), jnp.float32),
            pltpu.VMEM((_CHUNK, _DH), jnp.float32),
            pltpu.VMEM((_CHUNK, _CW), jnp.float32),
            pltpu.SemaphoreType.DMA,
            pltpu.SemaphoreType.DMA,
            pltpu.SemaphoreType.DMA,
            pltpu.SemaphoreType.DMA,
            pltpu.SemaphoreType.DMA,
            pltpu.SemaphoreType.DMA,
            pltpu.SemaphoreType.DMA,
            pltpu.SemaphoreType.DMA,
            pltpu.SemaphoreType.DMA,
        ],
        compiler_params=pltpu.CompilerParams(use_tc_tiling_on_sc=False),
    )
    return k(h, edges, zsum, zcnt, ones)


def _encode_body(x_ref, wi_ref, bi_ref, q_ref, wq_ref, bq_ref, o_ref):
    q = jnp.dot(q_ref[...], wq_ref[...],
                preferred_element_type=jnp.float32) + bq_ref[...]
    o_ref[...] = jnp.dot(x_ref[...], wi_ref[...],
                         preferred_element_type=jnp.float32) + bi_ref[...] + q


def _norm_residual(h, mean, wl_ref, bl_ref, wr_ref, g_ref, be_ref):
    hout = (jnp.dot(mean, wl_ref[...], preferred_element_type=jnp.float32)
            + bl_ref[...]
            + jnp.dot(h, wr_ref[...], preferred_element_type=jnp.float32))
    m = jnp.mean(hout, axis=-1, keepdims=True)
    d = hout - m
    var = jnp.mean(d * d, axis=-1, keepdims=True)
    y = d * lax.rsqrt(var + 1e-5) * g_ref[...] + be_ref[...]
    return h + y


def _layer0_body(h_ref, s_ref, cn_ref, wl_ref, bl_ref, wr_ref, g_ref,
                 be_ref, o_ref, c_ref):
    cnt = jnp.maximum((cn_ref[0] + cn_ref[1])[:, :1], 1.0)
    mean = (s_ref[0] + s_ref[1]) / cnt
    hn = _norm_residual(h_ref[...], mean, wl_ref, bl_ref, wr_ref, g_ref,
                        be_ref)
    o_ref[...] = jnp.maximum(hn, 0.0)
    c_ref[...] = cnt


def _final_body(h_ref, s_ref, c_ref, wl_ref, bl_ref, wr_ref, g_ref,
                be_ref, watt_ref, batt_ref, o_ref, a_ref):
    mean = (s_ref[0] + s_ref[1]) / c_ref[...]
    hn = _norm_residual(h_ref[...], mean, wl_ref, bl_ref, wr_ref, g_ref,
                        be_ref)
    o_ref[...] = hn
    logits = jnp.dot(hn, watt_ref[...],
                     preferred_element_type=jnp.float32) + batt_ref[...]
    z = logits - jnp.max(logits, axis=0, keepdims=True)
    e = jnp.exp(z)
    a_ref[...] = e / jnp.sum(e, axis=0, keepdims=True)


def kernel(x, edge_index, edge_attr, query_embedding, W_in, b_in, W_q, b_q,
           Wl0, bl0, Wr0, g0, be0, Wl1, bl1, Wr1, g1, be1, W_att, b_att):
    del edge_attr
    f32 = jnp.float32
    i32 = jnp.int32
    sds = jax.ShapeDtypeStruct

    npad = _EPAD - _E
    pad = jnp.concatenate(
        [jnp.zeros((1, npad), i32),
         (_N + (jnp.arange(npad, dtype=i32) % (_NA - _N)))[None]], axis=0)
    edges = jnp.concatenate([edge_index, pad],
                            axis=1).reshape(2, _NW, _NCHUNK, _CHUNK)

    h0 = pl.pallas_call(
        _encode_body, out_shape=sds((_N, _DH), f32))(
            x, W_in, b_in.reshape(1, _DH), query_embedding.reshape(1, -1),
            W_q, b_q.reshape(1, _DH))

    sums0, cnts = _sc_aggregate(h0, edges, with_cnt=True)

    h1, cnt = pl.pallas_call(
        _layer0_body, out_shape=(sds((_N, _DH), f32), sds((_N, 1), f32)))(
            h0, sums0, cnts, Wl0, bl0.reshape(1, -1), Wr0,
            g0.reshape(1, -1), be0.reshape(1, -1))

    sums1, _ = _sc_aggregate(h1, edges, with_cnt=False)

    h2, attn = pl.pallas_call(
        _final_body, out_shape=(sds((_N, _DH), f32), sds((_N, 1), f32)))(
            h1, sums1, cnt, Wl1, bl1.reshape(1, -1), Wr1,
            g1.reshape(1, -1), be1.reshape(1, -1), W_att,
            b_att.reshape(1, 1))

    return h2, attn.reshape(-1)


# speedup vs baseline: 2.0769x; 1.8782x over previous
import functools

import jax
import jax.numpy as jnp
from jax import lax
from jax.experimental import pallas as pl
from jax.experimental.pallas import tpu as pltpu
from jax.experimental.pallas import tpu_sc as plsc

_N = 10000
_E = 320000
_DH = 128
_CW = 16
_NC = 2
_NS = 16
_NW = _NC * _NS
_CHUNK = 80
_NCHUNK = 126
_EPW = _NCHUNK * _CHUNK
_EPAD = _NW * _EPW
_NA = _N + 16
_RPT = _N // _NS


def _sc_agg_body(with_cnt, h_hbm, edges_hbm, zsum_hbm, zcnt_hbm, ones_hbm,
                 sum_out, cnt_out, acc, cnt, idxb, rows0, rows1, rows2,
                 ones_v, g0, g1, g2, s0, s1, s2, r0s, r1s, r2s):
    cid = lax.axis_index("c")
    sid = lax.axis_index("s")
    wid = sid * _NC + cid
    row0 = sid * _RPT
    rows = (rows0, rows1, rows2)
    gsem = (g0, g1, g2)
    ssem = (s0, s1, s2)
    rsem = (r0s, r1s, r2s)
    T = _NCHUNK // 3

    def srow(tb, j):
        return idxb.at[tb, 0, j]

    def drow(tb, j):
        return idxb.at[tb, 1, j]

    def refill_start(t, tb):
        pltpu.make_async_copy(edges_hbm.at[0, wid, pl.ds(3 * t, 3)],
                              idxb.at[tb, 0], rsem[tb]).start()
        pltpu.make_async_copy(edges_hbm.at[1, wid, pl.ds(3 * t, 3)],
                              idxb.at[tb, 1], rsem[tb]).start()

    def refill_wait(t, tb):
        pltpu.make_async_copy(edges_hbm.at[0, wid, pl.ds(3 * t, 3)],
                              idxb.at[tb, 0], rsem[tb]).wait()
        pltpu.make_async_copy(edges_hbm.at[1, wid, pl.ds(3 * t, 3)],
                              idxb.at[tb, 1], rsem[tb]).wait()

    def step(buf, gw_row, sc_row, nx_row, first=False):
        pltpu.make_async_copy(h_hbm.at[gw_row], rows[buf], gsem[buf]).wait()
        pltpu.make_async_copy(rows[buf], acc.at[sc_row],
                              ssem[buf]).start(add=True)
        if with_cnt:
            pltpu.make_async_copy(ones_v, cnt.at[sc_row],
                                  ssem[buf]).start(add=True)
        pbuf = (buf - 1) % 3
        if not first:
            pltpu.make_async_copy(rows[pbuf], acc.at[sc_row],
                                  ssem[pbuf]).wait()
            if with_cnt:
                pltpu.make_async_copy(ones_v, cnt.at[sc_row],
                                      ssem[pbuf]).wait()
        if nx_row is not None:
            pltpu.make_async_copy(h_hbm.at[nx_row], rows[pbuf],
                                  gsem[pbuf]).start()

    pltpu.sync_copy(zsum_hbm.at[pl.ds(row0, _RPT)],
                    acc.at[pl.ds(row0, _RPT)])
    if with_cnt:
        pltpu.sync_copy(zcnt_hbm.at[pl.ds(row0, _RPT)],
                        cnt.at[pl.ds(row0, _RPT)])
        pltpu.sync_copy(ones_hbm, ones_v)
    for a in range(2):
        pltpu.sync_copy(edges_hbm.at[a, wid, pl.ds(0, 3)], idxb.at[0, a])
        pltpu.sync_copy(edges_hbm.at[a, wid, pl.ds(3, 3)], idxb.at[1, a])
    refill_start(2, 2)
    pltpu.async_copy(h_hbm.at[srow(0, 0)], rows0, g0)
    pltpu.async_copy(h_hbm.at[srow(0, 1)], rows1, g1)
    plsc.subcore_barrier()

    step(0, srow(0, 0), drow(0, 0), srow(0, 2), first=True)
    step(1, srow(0, 1), drow(0, 1), srow(1, 0))
    step(2, srow(0, 2), drow(0, 2), srow(1, 1))

    def triple(t, tb):
        nb = (tb + 1) % 3
        fb = (tb + 2) % 3
        refill_wait(t + 1, nb)
        step(0, srow(tb, 0), drow(tb, 0), srow(tb, 2))
        refill_start(t + 2, fb)
        step(1, srow(tb, 1), drow(tb, 1), srow(nb, 0))
        step(2, srow(tb, 2), drow(tb, 2), srow(nb, 1))

    @pl.loop(1, T - 2, step=3)
    def _(t):
        triple(t, 1)
        triple(t + 1, 2)
        triple(t + 2, 0)

    tb = (T - 2) % 3
    nb = (T - 1) % 3
    refill_wait(T - 1, nb)
    step(0, srow(tb, 0), drow(tb, 0), srow(tb, 2))
    step(1, srow(tb, 1), drow(tb, 1), srow(nb, 0))
    step(2, srow(tb, 2), drow(tb, 2), srow(nb, 1))
    step(0, srow(nb, 0), drow(nb, 0), srow(nb, 2))
    step(1, srow(nb, 1), drow(nb, 1), None)
    step(2, srow(nb, 2), drow(nb, 2), None)
    pltpu.make_async_copy(rows2, acc.at[drow(nb, 2)], ssem[2]).wait()
    if with_cnt:
        pltpu.make_async_copy(ones_v, cnt.at[drow(nb, 2)], ssem[2]).wait()

    plsc.subcore_barrier()
    pltpu.sync_copy(acc.at[pl.ds(row0, _RPT)],
                    sum_out.at[cid, pl.ds(row0, _RPT)])
    if with_cnt:
        pltpu.sync_copy(cnt.at[pl.ds(row0, _RPT)],
                        cnt_out.at[cid, pl.ds(row0, _RPT)])


def _sc_aggregate(h, edges, with_cnt):
    mesh = plsc.VectorSubcoreMesh(core_axis_name="c", subcore_axis_name="s")
    zsum = jnp.zeros((_N, _DH), jnp.float32)
    zcnt = jnp.zeros((_N, _CW), jnp.float32)
    ones = jnp.ones((_CHUNK, _CW), jnp.float32)
    k = pl.kernel(
        functools.partial(_sc_agg_body, with_cnt),
        out_type=(jax.ShapeDtypeStruct((_NC, _N, _DH), jnp.float32),
                  jax.ShapeDtypeStruct((_NC, _N, _CW), jnp.float32)),
        mesh=mesh,
        scratch_types=[
            pltpu.VMEM_SHARED((_NA, _DH), jnp.float32),
            pltpu.VMEM_SHARED((_NA, _CW) if with_cnt else (8, _CW),
                              jnp.float32),
            pltpu.VMEM((3, 2, 3, _CHUNK), jnp.int32),
            pltpu.VMEM((_CHUNK, _DH), jnp.float32),
            pltpu.VMEM((_CHUNK, _DH), jnp.float32),
            pltpu.VMEM((_CHUNK, _DH), jnp.float32),
            pltpu.VMEM((_CHUNK, _CW), jnp.float32),
            pltpu.SemaphoreType.DMA,
            pltpu.SemaphoreType.DMA,
            pltpu.SemaphoreType.DMA,
            pltpu.SemaphoreType.DMA,
            pltpu.SemaphoreType.DMA,
            pltpu.SemaphoreType.DMA,
            pltpu.SemaphoreType.DMA,
            pltpu.SemaphoreType.DMA,
            pltpu.SemaphoreType.DMA,
        ],
        compiler_params=pltpu.CompilerParams(use_tc_tiling_on_sc=False),
    )
    return k(h, edges, zsum, zcnt, ones)


def _encode_body(x_ref, wi_ref, bi_ref, q_ref, wq_ref, bq_ref, o_ref):
    q = jnp.dot(q_ref[...], wq_ref[...],
                preferred_element_type=jnp.float32) + bq_ref[...]
    o_ref[...] = jnp.dot(x_ref[...], wi_ref[...],
                         preferred_element_type=jnp.float32) + bi_ref[...] + q


def _norm_residual(h, mean, wl_ref, bl_ref, wr_ref, g_ref, be_ref):
    hout = (jnp.dot(mean, wl_ref[...], preferred_element_type=jnp.float32)
            + bl_ref[...]
            + jnp.dot(h, wr_ref[...], preferred_element_type=jnp.float32))
    m = jnp.mean(hout, axis=-1, keepdims=True)
    d = hout - m
    var = jnp.mean(d * d, axis=-1, keepdims=True)
    y = d * lax.rsqrt(var + 1e-5) * g_ref[...] + be_ref[...]
    return h + y


def _layer0_body(h_ref, s_ref, cn_ref, wl_ref, bl_ref, wr_ref, g_ref,
                 be_ref, o_ref, c_ref):
    cnt = jnp.maximum((cn_ref[0] + cn_ref[1])[:, :1], 1.0)
    mean = (s_ref[0] + s_ref[1]) / cnt
    hn = _norm_residual(h_ref[...], mean, wl_ref, bl_ref, wr_ref, g_ref,
                        be_ref)
    o_ref[...] = jnp.maximum(hn, 0.0)
    c_ref[...] = cnt


def _final_body(h_ref, s_ref, c_ref, wl_ref, bl_ref, wr_ref, g_ref,
                be_ref, watt_ref, batt_ref, o_ref, a_ref):
    mean = (s_ref[0] + s_ref[1]) / c_ref[...]
    hn = _norm_residual(h_ref[...], mean, wl_ref, bl_ref, wr_ref, g_ref,
                        be_ref)
    o_ref[...] = hn
    logits = jnp.dot(hn, watt_ref[...],
                     preferred_element_type=jnp.float32) + batt_ref[...]
    z = logits - jnp.max(logits, axis=0, keepdims=True)
    e = jnp.exp(z)
    a_ref[...] = e / jnp.sum(e, axis=0, keepdims=True)


def kernel(x, edge_index, edge_attr, query_embedding, W_in, b_in, W_q, b_q,
           Wl0, bl0, Wr0, g0, be0, Wl1, bl1, Wr1, g1, be1, W_att, b_att):
    del edge_attr
    f32 = jnp.float32
    i32 = jnp.int32
    sds = jax.ShapeDtypeStruct

    npad = _EPAD - _E
    iota = jnp.arange(npad, dtype=i32)
    pad = jnp.concatenate(
        [(iota * 37 % _N)[None], (_N + iota % (_NA - _N))[None]], axis=0)
    edges = jnp.concatenate([edge_index, pad],
                            axis=1).reshape(2, _NW, _NCHUNK, _CHUNK)

    h0 = pl.pallas_call(
        _encode_body, out_shape=sds((_N, _DH), f32))(
            x, W_in, b_in.reshape(1, _DH), query_embedding.reshape(1, -1),
            W_q, b_q.reshape(1, _DH))

    sums0, cnts = _sc_aggregate(h0, edges, with_cnt=True)

    h1, cnt = pl.pallas_call(
        _layer0_body, out_shape=(sds((_N, _DH), f32), sds((_N, 1), f32)))(
            h0, sums0, cnts, Wl0, bl0.reshape(1, -1), Wr0,
            g0.reshape(1, -1), be0.reshape(1, -1))

    sums1, _ = _sc_aggregate(h1, edges, with_cnt=False)

    h2, attn = pl.pallas_call(
        _final_body, out_shape=(sds((_N, _DH), f32), sds((_N, 1), f32)))(
            h1, sums1, cnt, Wl1, bl1.reshape(1, -1), Wr1,
            g1.reshape(1, -1), be1.reshape(1, -1), W_att,
            b_att.reshape(1, 1))

    return h2, attn.reshape(-1)
